# baseline (speedup 1.0000x reference)
import jax
import jax.numpy as jnp
from jax.experimental import pallas as pl
from jax.experimental.pallas import tpu as pltpu

_F32 = jnp.float32
_BF16 = jnp.bfloat16


def _body(xt_ref, oth_hbm, lng_ref, lnb_ref, gtile_ref, btile_ref, wih_ref,
          whh_ref, bih_ref, bhh_ref, wnb_ref, bnb_ref, wgatt_ref,
          asrc_ref, adst_ref, asrct_ref, bgat_ref, wq_ref, bq_ref, wk_ref,
          bk_ref, wv_ref, bv_ref, wo_ref, bo_ref, w1_ref, b1_ref, w2_ref,
          b2_ref, out_ref, othv_ref, xw_ref, gbig_ref, sem_o):
    T, B, F = xt_ref.shape
    NF = othv_ref.shape[2]
    N = NF // F
    H = whh_ref.shape[0]
    TB = T * B
    NH = N * H

    copies_o = [pltpu.make_async_copy(oth_hbm.at[:, pl.ds(t * NF, NF)],
                                      othv_ref.at[t], sem_o)
                for t in range(T)]
    for cp in copies_o:
        cp.start()

    def leaky(x):
        return jnp.where(x >= 0, x, 0.2 * x)

    def b16(x):
        return x.astype(_BF16)

    def dot16(a, b):
        return jnp.dot(b16(a), b16(b), preferred_element_type=_F32)

    def dotx(a, b):
        return jnp.dot(a, b, preferred_element_type=_F32,
                       precision=jax.lax.Precision.HIGHEST)

    x = xt_ref[:].reshape(TB, F)
    mu = jnp.mean(x, axis=-1, keepdims=True)
    xc = x - mu
    var = jnp.mean(xc * xc, axis=-1, keepdims=True)
    xn = xc * jax.lax.rsqrt(var + 1e-5) * lng_ref[:] + lnb_ref[:]
    xw_ref[:] = dot16(xn, wih_ref[:]) + bih_ref[:] + bhh_ref[:]

    whh16 = b16(whh_ref[:])
    h = jnp.zeros((B, H), _F32)
    c = jnp.zeros((B, H), _F32)
    hs = []
    for t in range(T):
        g4 = xw_ref[t * B:(t + 1) * B, :] + jnp.dot(
            b16(h), whh16, preferred_element_type=_F32)
        i_g = jax.nn.sigmoid(g4[:, 0:H])
        f_g = jax.nn.sigmoid(g4[:, H:2 * H])
        g_g = jnp.tanh(g4[:, 2 * H:3 * H])
        o_g = jax.nn.sigmoid(g4[:, 3 * H:4 * H])
        c = f_g * c + i_g * g_g
        h = o_g * jnp.tanh(c)
        hs.append(h)
    lstm = jnp.concatenate(hs, axis=0)

    gt = dot16(lstm, wgatt_ref[:])
    adst0 = dot16(gt, adst_ref[:])
    e_self = leaky(dot16(gt, asrc_ref[:]) + adst0)

    for cp in copies_o:
        cp.wait()
    ov = othv_ref[:].reshape(TB, NF)
    seg_row = jax.lax.broadcasted_iota(jnp.int32, (NF, N), 0) // F
    seg_col = jax.lax.broadcasted_iota(jnp.int32, (NF, N), 1)
    msum = jnp.where(seg_row == seg_col, 1.0 / F, 0.0)
    exp_row = jax.lax.broadcasted_iota(jnp.int32, (N, NF), 0)
    exp_col = jax.lax.broadcasted_iota(jnp.int32, (N, NF), 1) // F
    mexp = jnp.where(exp_row == exp_col, 1.0, 0.0)
    mu_o = dotx(ov, msum)
    s2_o = dotx(ov * ov, msum)
    istd_o = jax.lax.rsqrt(s2_o - mu_o * mu_o + 1e-5)
    on = (ov - dotx(mu_o, mexp)) * dotx(istd_o, mexp) * gtile_ref[:] \
        + btile_ref[:]

    wgatt16 = b16(wgatt_ref[:])
    for n in range(N):
        hn = jax.nn.relu(dot16(on[:, n * F:(n + 1) * F], wnb_ref[:])
                         + bnb_ref[:])
        gbig_ref[:, n * H:(n + 1) * H] = jnp.dot(
            b16(hn), wgatt16, preferred_element_type=_F32)

    blk_row = jax.lax.broadcasted_iota(jnp.int32, (NH, N), 0) // H
    blk_col = jax.lax.broadcasted_iota(jnp.int32, (NH, N), 1)
    a_blk = jnp.where(blk_row == blk_col, asrct_ref[:], 0.0)
    e_oth = leaky(dot16(gbig_ref[:], a_blk) + adst0)

    e = jnp.concatenate([e_oth, e_self], axis=1)
    m = jnp.max(e, axis=1, keepdims=True)
    p = jnp.exp(e - m)
    denom = jnp.sum(p, axis=1, keepdims=True)

    hexp_row = jax.lax.broadcasted_iota(jnp.int32, (N, NH), 0)
    hexp_col = jax.lax.broadcasted_iota(jnp.int32, (N, NH), 1) // H
    mhexp = jnp.where(hexp_row == hexp_col, 1.0, 0.0)
    p_big = dotx(p[:, 0:N], mhexp)
    acc = jnp.zeros((TB, H), _F32)
    for n in range(N):
        acc = acc + p_big[:, n * H:(n + 1) * H] * gbig_ref[:, n * H:(n + 1) * H]
    inv = 1.0 / denom
    combined = (acc + p[:, N:N + 1] * gt) * inv + bgat_ref[:]

    q = dot16(combined[(T - 1) * B:TB, :], wq_ref[:]) + bq_ref[:]
    k = dot16(combined, wk_ref[:]) + bk_ref[:]
    v = dot16(combined, wv_ref[:]) + bv_ref[:]
    q16 = b16(q).astype(_F32)
    k16 = b16(k).astype(_F32)
    scs = []
    for t in range(T):
        kt = k16[t * B:(t + 1) * B, :]
        scs.append(jnp.sum(q16 * kt, axis=1, keepdims=True))
    sc = jnp.concatenate(scs, axis=1) / jnp.sqrt(jnp.asarray(H, _F32))
    mt = jnp.max(sc, axis=1, keepdims=True)
    pt = jnp.exp(sc - mt)
    wt = pt / jnp.sum(pt, axis=1, keepdims=True)
    wt16 = b16(wt).astype(_F32)
    v16 = b16(v).astype(_F32)
    att = jnp.zeros((B, H), _F32)
    for t in range(T):
        att = att + wt16[:, t:t + 1] * v16[t * B:(t + 1) * B, :]
    att = dot16(att, wo_ref[:]) + bo_ref[:]

    hid = jax.nn.relu(dot16(att, w1_ref[:]) + b1_ref[:])
    out_ref[:] = dot16(hid, w2_ref[:]) + b2_ref[:]


def kernel(observed_trajectory_target, observed_trajectory_others, ln_g, ln_b,
           W_ih, W_hh, b_ih, b_hh, W_nb, b_nb, W_gat, a_src, a_dst, b_gat,
           Wq, bq, Wk, bk, Wv, bv, Wo, bo, W1, b1, W2, b2):
    B, T, F = observed_trajectory_target.shape
    N = observed_trajectory_others.shape[2]
    H = W_hh.shape[1]

    xt = jnp.transpose(observed_trajectory_target, (1, 0, 2))
    oth_flat = observed_trajectory_others.reshape(B, T * N * F)

    row = lambda v: v.reshape(1, -1)
    col = lambda v: v.reshape(-1, 1)
    hbm = pl.BlockSpec(memory_space=pltpu.MemorySpace.HBM)
    vmem = pl.BlockSpec(memory_space=pltpu.MemorySpace.VMEM)

    return pl.pallas_call(
        _body,
        out_shape=jax.ShapeDtypeStruct((B, 2), _F32),
        in_specs=[vmem, hbm] + [vmem] * 27,
        scratch_shapes=[
            pltpu.VMEM((T, B, N * F), _F32),
            pltpu.VMEM((T * B, 4 * H), _F32),
            pltpu.VMEM((T * B, N * H), _F32),
            pltpu.SemaphoreType.DMA,
        ],
    )(xt, oth_flat, row(ln_g), row(ln_b),
      row(jnp.tile(ln_g, N)), row(jnp.tile(ln_b, N)), W_ih.T, W_hh.T,
      row(b_ih), row(b_hh), W_nb.T, row(b_nb), W_gat.T, col(a_src),
      col(a_dst), col(jnp.tile(a_src, N)), row(b_gat), Wq.T, row(bq),
      Wk.T, row(bk), Wv.T, row(bv), Wo.T, row(bo), W1.T, row(b1), W2.T,
      row(b2))


# speedup vs baseline: 1.2958x; 1.2958x over previous
import jax
import jax.numpy as jnp
from jax.experimental import pallas as pl
from jax.experimental.pallas import tpu as pltpu

_F32 = jnp.float32
_BF16 = jnp.bfloat16


def _body(xt_ref, oth_hbm, lng_ref, lnb_ref, gtile_ref, btile_ref, wih_ref,
          whh_ref, bih_ref, bhh_ref, wnb_ref, bnb_ref, wgatt_ref,
          asrc_ref, adst_ref, asrct_ref, bgat_ref, wq_ref, bq_ref, wk_ref,
          bk_ref, wv_ref, bv_ref, wo_ref, bo_ref, w1_ref, b1_ref, w2_ref,
          b2_ref, out_ref, othv_ref, xw_ref, gbig_ref, sem_o):
    T, B, F = xt_ref.shape
    NF = othv_ref.shape[2]
    N = NF // F
    H = whh_ref.shape[0]
    TB = T * B
    NH = N * H

    copies_o = [pltpu.make_async_copy(oth_hbm.at[:, pl.ds(t * NF, NF)],
                                      othv_ref.at[t], sem_o)
                for t in range(T)]
    for cp in copies_o:
        cp.start()

    def leaky(x):
        return jnp.where(x >= 0, x, 0.2 * x)

    def b16(x):
        return x.astype(_BF16)

    def dot16(a, b):
        return jnp.dot(b16(a), b16(b), preferred_element_type=_F32)

    def dotx(a, b):
        return jnp.dot(a, b, preferred_element_type=_F32)

    x = xt_ref[:].reshape(TB, F)
    mu = jnp.mean(x, axis=-1, keepdims=True)
    xc = x - mu
    var = jnp.mean(xc * xc, axis=-1, keepdims=True)
    xn = xc * jax.lax.rsqrt(var + 1e-5) * lng_ref[:] + lnb_ref[:]
    xw_ref[:] = dot16(xn, wih_ref[:]) + bih_ref[:] + bhh_ref[:]

    whh16 = b16(whh_ref[:])
    h = jnp.zeros((B, H), _F32)
    c = jnp.zeros((B, H), _F32)
    hs = []
    for t in range(T):
        g4 = xw_ref[t * B:(t + 1) * B, :] + jnp.dot(
            b16(h), whh16, preferred_element_type=_F32)
        i_g = jax.nn.sigmoid(g4[:, 0:H])
        f_g = jax.nn.sigmoid(g4[:, H:2 * H])
        g_g = jnp.tanh(g4[:, 2 * H:3 * H])
        o_g = jax.nn.sigmoid(g4[:, 3 * H:4 * H])
        c = f_g * c + i_g * g_g
        h = o_g * jnp.tanh(c)
        hs.append(h)
    lstm = jnp.concatenate(hs, axis=0)

    gt = dot16(lstm, wgatt_ref[:])
    gt16 = b16(gt)
    adst0 = jnp.dot(gt16, b16(adst_ref[:]),
                    preferred_element_type=_F32)
    e_self = leaky(jnp.dot(gt16, b16(asrc_ref[:]),
                           preferred_element_type=_F32) + adst0)

    for cp in copies_o:
        cp.wait()
    ov = othv_ref[:].reshape(TB, NF)
    seg_row = jax.lax.broadcasted_iota(jnp.int32, (NF, N), 0) // F
    seg_col = jax.lax.broadcasted_iota(jnp.int32, (NF, N), 1)
    msum = jnp.where(seg_row == seg_col, 1.0 / F, 0.0)
    exp_row = jax.lax.broadcasted_iota(jnp.int32, (N, NF), 0)
    exp_col = jax.lax.broadcasted_iota(jnp.int32, (N, NF), 1) // F
    mexp = jnp.where(exp_row == exp_col, 1.0, 0.0)
    mu_o = dotx(ov, msum)
    s2_o = dotx(ov * ov, msum)
    istd_o = jax.lax.rsqrt(s2_o - mu_o * mu_o + 1e-5)
    on = (ov - dotx(mu_o, mexp)) * dotx(istd_o, mexp) * gtile_ref[:] \
        + btile_ref[:]

    wgatt16 = b16(wgatt_ref[:])
    for n in range(N):
        hn = jax.nn.relu(dot16(on[:, n * F:(n + 1) * F], wnb_ref[:])
                         + bnb_ref[:])
        gbig_ref[:, n * H:(n + 1) * H] = jnp.dot(
            b16(hn), wgatt16, preferred_element_type=_F32)

    blk_row = jax.lax.broadcasted_iota(jnp.int32, (NH, N), 0) // H
    blk_col = jax.lax.broadcasted_iota(jnp.int32, (NH, N), 1)
    a_blk = jnp.where(blk_row == blk_col, asrct_ref[:], 0.0)
    e_oth = leaky(dot16(gbig_ref[:], a_blk) + adst0)

    e = jnp.concatenate([e_oth, e_self], axis=1)
    m = jnp.max(e, axis=1, keepdims=True)
    p = jnp.exp(e - m)
    denom = jnp.sum(p, axis=1, keepdims=True)

    hexp_row = jax.lax.broadcasted_iota(jnp.int32, (N, NH), 0)
    hexp_col = jax.lax.broadcasted_iota(jnp.int32, (N, NH), 1) // H
    mhexp = jnp.where(hexp_row == hexp_col, 1.0, 0.0)
    p_big = dotx(p[:, 0:N], mhexp)
    acc = jnp.zeros((TB, H), _F32)
    for n in range(N):
        acc = acc + p_big[:, n * H:(n + 1) * H] * gbig_ref[:, n * H:(n + 1) * H]
    inv = 1.0 / denom
    combined = (acc + p[:, N:N + 1] * gt) * inv + bgat_ref[:]

    c16 = b16(combined)
    q = jnp.dot(c16[(T - 1) * B:TB, :], b16(wq_ref[:]),
                preferred_element_type=_F32) + bq_ref[:]
    k = jnp.dot(c16, b16(wk_ref[:]), preferred_element_type=_F32) + bk_ref[:]
    v = jnp.dot(c16, b16(wv_ref[:]), preferred_element_type=_F32) + bv_ref[:]
    q16 = b16(q).astype(_F32)
    k16 = b16(k).astype(_F32)
    scs = []
    for t in range(T):
        kt = k16[t * B:(t + 1) * B, :]
        scs.append(jnp.sum(q16 * kt, axis=1, keepdims=True))
    sc = jnp.concatenate(scs, axis=1) / jnp.sqrt(jnp.asarray(H, _F32))
    mt = jnp.max(sc, axis=1, keepdims=True)
    pt = jnp.exp(sc - mt)
    wt = pt / jnp.sum(pt, axis=1, keepdims=True)
    wt16 = b16(wt).astype(_F32)
    v16 = b16(v).astype(_F32)
    att = jnp.zeros((B, H), _F32)
    for t in range(T):
        att = att + wt16[:, t:t + 1] * v16[t * B:(t + 1) * B, :]
    att = dot16(att, wo_ref[:]) + bo_ref[:]

    hid = jax.nn.relu(dot16(att, w1_ref[:]) + b1_ref[:])
    out_ref[:] = dot16(hid, w2_ref[:]) + b2_ref[:]


def kernel(observed_trajectory_target, observed_trajectory_others, ln_g, ln_b,
           W_ih, W_hh, b_ih, b_hh, W_nb, b_nb, W_gat, a_src, a_dst, b_gat,
           Wq, bq, Wk, bk, Wv, bv, Wo, bo, W1, b1, W2, b2):
    B, T, F = observed_trajectory_target.shape
    N = observed_trajectory_others.shape[2]
    H = W_hh.shape[1]

    xt = jnp.transpose(observed_trajectory_target, (1, 0, 2))
    oth_flat = observed_trajectory_others.reshape(B, T * N * F)

    row = lambda v: v.reshape(1, -1)
    col = lambda v: v.reshape(-1, 1)
    hbm = pl.BlockSpec(memory_space=pltpu.MemorySpace.HBM)
    vmem = pl.BlockSpec(memory_space=pltpu.MemorySpace.VMEM)

    return pl.pallas_call(
        _body,
        out_shape=jax.ShapeDtypeStruct((B, 2), _F32),
        in_specs=[vmem, hbm] + [vmem] * 27,
        scratch_shapes=[
            pltpu.VMEM((T, B, N * F), _F32),
            pltpu.VMEM((T * B, 4 * H), _F32),
            pltpu.VMEM((T * B, N * H), _F32),
            pltpu.SemaphoreType.DMA,
        ],
    )(xt, oth_flat, row(ln_g), row(ln_b),
      row(jnp.tile(ln_g, N)), row(jnp.tile(ln_b, N)), W_ih.T, W_hh.T,
      row(b_ih), row(b_hh), W_nb.T, row(b_nb), W_gat.T, col(a_src),
      col(a_dst), col(jnp.tile(a_src, N)), row(b_gat), Wq.T, row(bq),
      Wk.T, row(bk), Wv.T, row(bv), Wo.T, row(bo), W1.T, row(b1), W2.T,
      row(b2))


# speedup vs baseline: 1.5465x; 1.1934x over previous
import jax
import jax.numpy as jnp
from jax.experimental import pallas as pl
from jax.experimental.pallas import tpu as pltpu

_F32 = jnp.float32
_BF16 = jnp.bfloat16


def _body(xt_ref, oth_hbm, lng_ref, lnb_ref, wih_ref, whh_ref, bih_ref,
          bhh_ref, wnb_ref, bnb_ref, wgat_ref, asrc_ref, adst_ref, bgat_ref,
          wq_ref, bq_ref, wk_ref, bk_ref, wv_ref, bv_ref, wo_ref, bo_ref,
          w1_ref, b1_ref, w2_ref, b2_ref, out_ref,
          othv_ref, xw_ref, gbig_ref, sem_o):
    T, B, F = xt_ref.shape
    NF = othv_ref.shape[2]
    N = NF // F
    H = whh_ref.shape[1]
    TB = T * B
    NH = N * H

    copies_o = [pltpu.make_async_copy(oth_hbm.at[:, pl.ds(t * NF, NF)],
                                      othv_ref.at[t], sem_o)
                for t in range(T)]
    for cp in copies_o:
        cp.start()

    def leaky(x):
        return jnp.where(x >= 0, x, 0.2 * x)

    def b16(x):
        return x.astype(_BF16)

    def dot16(a, bt):
        return jnp.dot(b16(a), bt, preferred_element_type=_F32)

    def t16(w):
        return b16(jnp.transpose(w))

    def dotx(a, b):
        b_ = b16(b)
        a0 = b16(a)
        r0 = a - a0.astype(_F32)
        a1 = b16(r0)
        a2 = b16(r0 - a1.astype(_F32))
        out = jnp.dot(a0, b_, preferred_element_type=_F32)
        out = out + jnp.dot(a1, b_, preferred_element_type=_F32)
        return out + jnp.dot(a2, b_, preferred_element_type=_F32)

    x = xt_ref[:].reshape(TB, F)
    mu = jnp.mean(x, axis=-1, keepdims=True)
    xc = x - mu
    var = jnp.mean(xc * xc, axis=-1, keepdims=True)
    xn = xc * jax.lax.rsqrt(var + 1e-5) * lng_ref[:] + lnb_ref[:]
    xw_ref[:] = dot16(xn, t16(wih_ref[:])) + bih_ref[:] + bhh_ref[:]

    whh16 = t16(whh_ref[:])
    h = jnp.zeros((B, H), _F32)
    c = jnp.zeros((B, H), _F32)
    hs = []
    for t in range(T):
        g4 = xw_ref[t * B:(t + 1) * B, :] + jnp.dot(
            b16(h), whh16, preferred_element_type=_F32)
        i_g = jax.nn.sigmoid(g4[:, 0:H])
        f_g = jax.nn.sigmoid(g4[:, H:2 * H])
        g_g = jnp.tanh(g4[:, 2 * H:3 * H])
        o_g = jax.nn.sigmoid(g4[:, 3 * H:4 * H])
        c = f_g * c + i_g * g_g
        h = o_g * jnp.tanh(c)
        hs.append(h)
    lstm = jnp.concatenate(hs, axis=0)

    wgatt16 = t16(wgat_ref[:])
    gt = dot16(lstm, wgatt16)
    gt16 = b16(gt)
    asrc_col = jnp.transpose(asrc_ref[:])
    adst_col = jnp.transpose(adst_ref[:])
    adst0 = jnp.dot(gt16, b16(adst_col),
                    preferred_element_type=_F32)
    e_self = leaky(jnp.dot(gt16, b16(asrc_col),
                           preferred_element_type=_F32) + adst0)

    for cp in copies_o:
        cp.wait()
    ov = othv_ref[:].reshape(TB, NF)
    seg_row = jax.lax.broadcasted_iota(jnp.int32, (NF, N), 0) // F
    seg_col = jax.lax.broadcasted_iota(jnp.int32, (NF, N), 1)
    msum = jnp.where(seg_row == seg_col, 1.0 / F, 0.0)
    exp_row = jax.lax.broadcasted_iota(jnp.int32, (N, NF), 0)
    exp_col = jax.lax.broadcasted_iota(jnp.int32, (N, NF), 1) // F
    mexp = jnp.where(exp_row == exp_col, 1.0, 0.0)
    mu_o = dotx(ov, msum)
    s2_o = dotx(ov * ov, msum)
    istd_o = jax.lax.rsqrt(s2_o - mu_o * mu_o + 1e-5)
    gtile = jnp.concatenate([lng_ref[:]] * N, axis=1)
    btile = jnp.concatenate([lnb_ref[:]] * N, axis=1)
    on = (ov - dotx(mu_o, mexp)) * dotx(istd_o, mexp) * gtile + btile

    wnbt16 = t16(wnb_ref[:])
    for n in range(N):
        hn = jax.nn.relu(dot16(on[:, n * F:(n + 1) * F], wnbt16)
                         + bnb_ref[:])
        gbig_ref[:, n * H:(n + 1) * H] = jnp.dot(
            b16(hn), wgatt16, preferred_element_type=_F32)

    tile_r = jax.lax.broadcasted_iota(jnp.int32, (NH, H), 0) % H
    tile_c = jax.lax.broadcasted_iota(jnp.int32, (NH, H), 1)
    mtile = jnp.where(tile_r == tile_c, 1.0, 0.0)
    asrct = dotx(mtile, asrc_col)
    blk_row = jax.lax.broadcasted_iota(jnp.int32, (NH, N), 0) // H
    blk_col = jax.lax.broadcasted_iota(jnp.int32, (NH, N), 1)
    a_blk = jnp.where(blk_row == blk_col, asrct, 0.0)
    e_oth = leaky(jnp.dot(b16(gbig_ref[:]), b16(a_blk),
                          preferred_element_type=_F32) + adst0)

    e = jnp.concatenate([e_oth, e_self], axis=1)
    m = jnp.max(e, axis=1, keepdims=True)
    p = jnp.exp(e - m)
    denom = jnp.sum(p, axis=1, keepdims=True)

    hexp_row = jax.lax.broadcasted_iota(jnp.int32, (N, NH), 0)
    hexp_col = jax.lax.broadcasted_iota(jnp.int32, (N, NH), 1) // H
    mhexp = jnp.where(hexp_row == hexp_col, 1.0, 0.0)
    p_big = dotx(p[:, 0:N], mhexp)
    acc = jnp.zeros((TB, H), _F32)
    for n in range(N):
        acc = acc + p_big[:, n * H:(n + 1) * H] * gbig_ref[:, n * H:(n + 1) * H]
    inv = 1.0 / denom
    combined = (acc + p[:, N:N + 1] * gt) * inv + bgat_ref[:]

    c16 = b16(combined)
    q = jnp.dot(c16[(T - 1) * B:TB, :], t16(wq_ref[:]),
                preferred_element_type=_F32) + bq_ref[:]
    k = jnp.dot(c16, t16(wk_ref[:]), preferred_element_type=_F32) + bk_ref[:]
    v = jnp.dot(c16, t16(wv_ref[:]), preferred_element_type=_F32) + bv_ref[:]
    q16 = b16(q).astype(_F32)
    k16 = b16(k).astype(_F32)
    scs = []
    for t in range(T):
        kt = k16[t * B:(t + 1) * B, :]
        scs.append(jnp.sum(q16 * kt, axis=1, keepdims=True))
    sc = jnp.concatenate(scs, axis=1) / jnp.sqrt(jnp.asarray(H, _F32))
    mt = jnp.max(sc, axis=1, keepdims=True)
    pt = jnp.exp(sc - mt)
    wt = pt / jnp.sum(pt, axis=1, keepdims=True)
    wt16 = b16(wt).astype(_F32)
    v16 = b16(v).astype(_F32)
    att = jnp.zeros((B, H), _F32)
    for t in range(T):
        att = att + wt16[:, t:t + 1] * v16[t * B:(t + 1) * B, :]
    att = dot16(att, t16(wo_ref[:])) + bo_ref[:]

    hid = jax.nn.relu(dot16(att, t16(w1_ref[:])) + b1_ref[:])
    out_ref[:] = dot16(hid, t16(w2_ref[:])) + b2_ref[:]


def kernel(observed_trajectory_target, observed_trajectory_others, ln_g, ln_b,
           W_ih, W_hh, b_ih, b_hh, W_nb, b_nb, W_gat, a_src, a_dst, b_gat,
           Wq, bq, Wk, bk, Wv, bv, Wo, bo, W1, b1, W2, b2):
    B, T, F = observed_trajectory_target.shape
    N = observed_trajectory_others.shape[2]
    H = W_hh.shape[1]

    xt = jnp.transpose(observed_trajectory_target, (1, 0, 2))
    oth_flat = observed_trajectory_others.reshape(B, T * N * F)

    row = lambda v: v.reshape(1, -1)
    hbm = pl.BlockSpec(memory_space=pltpu.MemorySpace.HBM)
    vmem = pl.BlockSpec(memory_space=pltpu.MemorySpace.VMEM)

    return pl.pallas_call(
        _body,
        out_shape=jax.ShapeDtypeStruct((B, 2), _F32),
        in_specs=[vmem, hbm] + [vmem] * 24,
        scratch_shapes=[
            pltpu.VMEM((T, B, N * F), _F32),
            pltpu.VMEM((T * B, 4 * H), _F32),
            pltpu.VMEM((T * B, N * H), _F32),
            pltpu.SemaphoreType.DMA,
        ],
    )(xt, oth_flat, row(ln_g), row(ln_b), W_ih, W_hh, row(b_ih), row(b_hh),
      W_nb, row(b_nb), W_gat, row(a_src), row(a_dst), row(b_gat),
      Wq, row(bq), Wk, row(bk), Wv, row(bv), Wo, row(bo), W1, row(b1),
      W2, row(b2))
